# Initial kernel scaffold; baseline (speedup 1.0000x reference)
import jax
import jax.numpy as jnp
from jax.experimental import pallas as pl


def kernel(h_V, h_E, E_idx, W1, b1, W2, b2, W3, b3, W11, b11, W12, b12, W13, b13, Wi, bi, Wo, bo, s1, o1, s2, o2, s3, o3):
    raise NotImplementedError("write your pallas kernel here")


# speedup vs baseline: 4.0421x; 4.0421x over previous
import functools

import jax
import jax.numpy as jnp
from jax import lax
from jax.experimental import pallas as pl
from jax.experimental.pallas import tpu as pltpu
from jax.experimental.pallas import tpu_sc as plsc

_N = 10000
_K = 32
_H = 128
_SCALE = 30.0

_NC = 2
_NS = 16
_NW = _NC * _NS
_B = _N * _K
_BPW = _B // _NW
_CH = 80
_NCHUNK = _BPW // _CH


def _gelu(x):
    return 0.5 * x * (1.0 + lax.erf(x * (2.0 ** -0.5)))


def _ln(x, s, o):
    m = jnp.mean(x, axis=-1, keepdims=True)
    v = jnp.mean((x - m) * (x - m), axis=-1, keepdims=True)
    return s * (x - m) * lax.rsqrt(v + 1e-5) + o


def _dot(a, b):
    return jnp.dot(a, b, preferred_element_type=jnp.float32)


def _sc_gather(table, idx_flat):
    mesh = plsc.VectorSubcoreMesh(core_axis_name="c", subcore_axis_name="s")

    @functools.partial(
        pl.kernel,
        mesh=mesh,
        out_type=jax.ShapeDtypeStruct((_B, _H), jnp.float32),
        scratch_types=[
            pltpu.VMEM((_CH,), jnp.int32),
            pltpu.VMEM((_CH, _H), jnp.float32),
            pltpu.SemaphoreType.DMA,
        ],
    )
    def k(table_hbm, idx_hbm, out_hbm, idx_v, rows_v, sem):
        wid = lax.axis_index("s") * _NC + lax.axis_index("c")
        base = wid * _BPW

        def body(t, carry):
            off = base + t * _CH
            pltpu.sync_copy(idx_hbm.at[pl.ds(off, _CH)], idx_v)
            pltpu.async_copy(table_hbm.at[idx_v], rows_v, sem).wait()
            pltpu.sync_copy(rows_v, out_hbm.at[pl.ds(off, _CH)])
            return carry

        lax.fori_loop(0, _NCHUNK, body, 0)

    return k(table, idx_flat)


def _pre_kernel(hv_ref, w_ref, out_ref):
    out_ref[...] = _dot(hv_ref[...], w_ref[...])


def _pre(h_V, W1c):
    bn = 1000
    return pl.pallas_call(
        _pre_kernel,
        grid=(_N // bn,),
        in_specs=[
            pl.BlockSpec((bn, _H), lambda i: (i, 0)),
            pl.BlockSpec((_H, _H), lambda i: (0, 0)),
        ],
        out_specs=pl.BlockSpec((bn, _H), lambda i: (i, 0)),
        out_shape=jax.ShapeDtypeStruct((_N, _H), jnp.float32),
    )(h_V, W1c)


_BN = 200


def _passA_kernel(he_ref, g1_ref, hv_ref, w1a_ref, b1_ref, w1b_ref, w2_ref,
                  b2_ref, w3_ref, b3_ref, s1_ref, o1_ref, wi_ref, bi_ref,
                  wo_ref, bo_ref, s2_ref, o2_ref, w11a_ref, b11_ref, w11c_ref,
                  hv2_ref, c2_ref, a2_ref):
    hv = hv_ref[...]
    a1 = _dot(hv, w1a_ref[...]) + b1_ref[...]
    he = he_ref[...].reshape(_BN * _K, _H)
    x = g1_ref[...] + _dot(he, w1b_ref[...])
    x = x.reshape(_BN, _K, _H) + a1[:, None, :]
    x = x.reshape(_BN * _K, _H)
    x = _dot(_gelu(x), w2_ref[...]) + b2_ref[...]
    x = _dot(_gelu(x), w3_ref[...]) + b3_ref[...]
    dh = jnp.sum(x.reshape(_BN, _K, _H), axis=1) * (1.0 / _SCALE)
    v = _ln(hv + dh, s1_ref[...], o1_ref[...])
    f = _dot(_gelu(_dot(v, wi_ref[...]) + bi_ref[...]), wo_ref[...]) + bo_ref[...]
    v2 = _ln(v + f, s2_ref[...], o2_ref[...])
    hv2_ref[...] = v2
    c2_ref[...] = _dot(v2, w11c_ref[...])
    a2_ref[...] = _dot(v2, w11a_ref[...]) + b11_ref[...]


def _passA(h_E, g1, h_V, W1a, b1, W1b, W2, b2, W3, b3, s1, o1, Wi, bi, Wo, bo,
           s2, o2, W11a, b11, W11c):
    nb = _N // _BN
    row = lambda i: (0, 0)
    sq = lambda i: (0, 0)
    out_sds = jax.ShapeDtypeStruct((_N, _H), jnp.float32)
    return pl.pallas_call(
        _passA_kernel,
        grid=(nb,),
        in_specs=[
            pl.BlockSpec((_BN, _K, _H), lambda i: (i, 0, 0)),
            pl.BlockSpec((_BN * _K, _H), lambda i: (i, 0)),
            pl.BlockSpec((_BN, _H), lambda i: (i, 0)),
            pl.BlockSpec((_H, _H), sq),
            pl.BlockSpec((1, _H), row),
            pl.BlockSpec((_H, _H), sq),
            pl.BlockSpec((_H, _H), sq),
            pl.BlockSpec((1, _H), row),
            pl.BlockSpec((_H, _H), sq),
            pl.BlockSpec((1, _H), row),
            pl.BlockSpec((1, _H), row),
            pl.BlockSpec((1, _H), row),
            pl.BlockSpec((_H, 4 * _H), sq),
            pl.BlockSpec((1, 4 * _H), row),
            pl.BlockSpec((4 * _H, _H), sq),
            pl.BlockSpec((1, _H), row),
            pl.BlockSpec((1, _H), row),
            pl.BlockSpec((1, _H), row),
            pl.BlockSpec((_H, _H), sq),
            pl.BlockSpec((1, _H), row),
            pl.BlockSpec((_H, _H), sq),
        ],
        out_specs=[
            pl.BlockSpec((_BN, _H), lambda i: (i, 0)),
            pl.BlockSpec((_BN, _H), lambda i: (i, 0)),
            pl.BlockSpec((_BN, _H), lambda i: (i, 0)),
        ],
        out_shape=[out_sds, out_sds, out_sds],
    )(h_E, g1, h_V, W1a, b1, W1b, W2, b2, W3, b3, s1, o1, Wi, bi, Wo, bo,
      s2, o2, W11a, b11, W11c)


def _passB_kernel(he_ref, g2_ref, a2_ref, w11b_ref, w12_ref, b12_ref, w13_ref,
                  b13_ref, s3_ref, o3_ref, out_ref):
    he = he_ref[...].reshape(_BN * _K, _H)
    x = g2_ref[...] + _dot(he, w11b_ref[...])
    x = x.reshape(_BN, _K, _H) + a2_ref[...][:, None, :]
    x = x.reshape(_BN * _K, _H)
    x = _dot(_gelu(x), w12_ref[...]) + b12_ref[...]
    x = _dot(_gelu(x), w13_ref[...]) + b13_ref[...]
    y = _ln(he + x, s3_ref[...], o3_ref[...])
    out_ref[...] = y.reshape(_BN, _K, _H)


def _passB(h_E, g2, a2, W11b, W12, b12, W13, b13, s3, o3):
    nb = _N // _BN
    row = lambda i: (0, 0)
    sq = lambda i: (0, 0)
    return pl.pallas_call(
        _passB_kernel,
        grid=(nb,),
        in_specs=[
            pl.BlockSpec((_BN, _K, _H), lambda i: (i, 0, 0)),
            pl.BlockSpec((_BN * _K, _H), lambda i: (i, 0)),
            pl.BlockSpec((_BN, _H), lambda i: (i, 0)),
            pl.BlockSpec((_H, _H), sq),
            pl.BlockSpec((_H, _H), sq),
            pl.BlockSpec((1, _H), row),
            pl.BlockSpec((_H, _H), sq),
            pl.BlockSpec((1, _H), row),
            pl.BlockSpec((1, _H), row),
            pl.BlockSpec((1, _H), row),
        ],
        out_specs=pl.BlockSpec((_BN, _K, _H), lambda i: (i, 0, 0)),
        out_shape=jax.ShapeDtypeStruct((_N, _K, _H), jnp.float32),
    )(h_E, g2, a2, W11b, W12, b12, W13, b13, s3, o3)


def kernel(h_V, h_E, E_idx, W1, b1, W2, b2, W3, b3, W11, b11, W12, b12, W13,
           b13, Wi, bi, Wo, bo, s1, o1, s2, o2, s3, o3):
    r = lambda v: v.reshape(1, -1)
    W1a, W1b, W1c = W1[:_H], W1[_H:2 * _H], W1[2 * _H:]
    W11a, W11b, W11c = W11[:_H], W11[_H:2 * _H], W11[2 * _H:]
    e_flat = E_idx.reshape(-1)

    c1 = _pre(h_V, W1c)
    g1 = _sc_gather(c1, e_flat)
    h_V2, c2, a2 = _passA(h_E, g1, h_V, W1a, r(b1), W1b, W2, r(b2), W3, r(b3),
                          r(s1), r(o1), Wi, r(bi), Wo, r(bo), r(s2), r(o2),
                          W11a, r(b11), W11c)
    g2 = _sc_gather(c2, e_flat)
    h_E_out = _passB(h_E, g2, a2, W11b, W12, r(b12), W13, r(b13), r(s3), r(o3))
    return (h_V2, h_E_out)


# speedup vs baseline: 5.8997x; 1.4596x over previous
import functools

import jax
import jax.numpy as jnp
from jax import lax
from jax.experimental import pallas as pl
from jax.experimental.pallas import tpu as pltpu
from jax.experimental.pallas import tpu_sc as plsc

_N = 10000
_K = 32
_H = 128
_SCALE = 30.0

_NC = 2
_NS = 16
_NW = _NC * _NS
_B = _N * _K
_BPW = _B // _NW
_CH = 80
_NCHUNK = _BPW // _CH


def _gelu(x):
    return 0.5 * x * (1.0 + lax.erf(x * (2.0 ** -0.5)))


def _ln(x, s, o):
    m = jnp.mean(x, axis=-1, keepdims=True)
    v = jnp.mean((x - m) * (x - m), axis=-1, keepdims=True)
    return s * (x - m) * lax.rsqrt(v + 1e-5) + o


def _dot(a, b):
    return jnp.dot(a, b, preferred_element_type=jnp.float32)


_CPG = 5
_GR = _CPG * _CH
_NG = _BPW // _GR
_NGM = _NG - 1


def _sc_gather(table, idx_flat):
    mesh = plsc.VectorSubcoreMesh(core_axis_name="c", subcore_axis_name="s")

    @functools.partial(
        pl.kernel,
        mesh=mesh,
        out_type=jax.ShapeDtypeStruct((_B, _H), jnp.float32),
        scratch_types=[
            pltpu.VMEM((_BPW,), jnp.int32),
            pltpu.VMEM((2, _CPG, _CH, _H), jnp.float32),
            pltpu.SemaphoreType.DMA,
            pltpu.SemaphoreType.DMA,
            pltpu.SemaphoreType.DMA,
            pltpu.SemaphoreType.DMA,
        ],
    )
    def k(table_hbm, idx_hbm, out_hbm, idx_v, rows_v, gs0, gs1, ss0, ss1):
        wid = lax.axis_index("s") * _NC + lax.axis_index("c")
        base = wid * _BPW
        gsem = (gs0, gs1)
        ssem = (ss0, ss1)

        pltpu.sync_copy(idx_hbm.at[pl.ds(base, _BPW)], idx_v)

        def do_group(g, p, first_pair):
            def drain_stores():
                for j in range(_CPG):
                    pltpu.make_async_copy(
                        rows_v.at[p, j],
                        out_hbm.at[pl.ds(base, _CH)],
                        ssem[p],
                    ).wait()

            if first_pair is None:
                drain_stores()
            else:
                pl.when(jnp.logical_not(first_pair))(drain_stores)

            goff = g * _GR
            copies = []
            for j in range(_CPG):
                copies.append(pltpu.async_copy(
                    table_hbm.at[idx_v.at[pl.ds(goff + j * _CH, _CH)]],
                    rows_v.at[p, j],
                    gsem[p],
                ))
            for c in copies:
                c.wait()
            for j in range(_CPG):
                pltpu.async_copy(
                    rows_v.at[p, j],
                    out_hbm.at[pl.ds(base + goff + j * _CH, _CH)],
                    ssem[p],
                )

        def pair(g2, carry):
            first = g2 == 0
            do_group(2 * g2, 0, first)
            do_group(2 * g2 + 1, 1, first)
            return carry

        lax.fori_loop(0, _NGM // 2, pair, 0)
        do_group(_NGM, 0, None)

        for p in (1, 0):
            for j in range(_CPG):
                pltpu.make_async_copy(
                    rows_v.at[p, j],
                    out_hbm.at[pl.ds(base, _CH)],
                    ssem[p],
                ).wait()

    return k(table, idx_flat)


def _pre_kernel(hv_ref, w_ref, out_ref):
    out_ref[...] = _dot(hv_ref[...], w_ref[...])


def _pre(h_V, W1c):
    bn = 1000
    return pl.pallas_call(
        _pre_kernel,
        grid=(_N // bn,),
        in_specs=[
            pl.BlockSpec((bn, _H), lambda i: (i, 0)),
            pl.BlockSpec((_H, _H), lambda i: (0, 0)),
        ],
        out_specs=pl.BlockSpec((bn, _H), lambda i: (i, 0)),
        out_shape=jax.ShapeDtypeStruct((_N, _H), jnp.float32),
    )(h_V, W1c)


_BN = 200


def _passA_kernel(he_ref, g1_ref, hv_ref, w1a_ref, b1_ref, w1b_ref, w2_ref,
                  b2_ref, w3_ref, b3_ref, s1_ref, o1_ref, wi_ref, bi_ref,
                  wo_ref, bo_ref, s2_ref, o2_ref, w11a_ref, b11_ref, w11c_ref,
                  hv2_ref, c2_ref, a2_ref):
    hv = hv_ref[...]
    a1 = _dot(hv, w1a_ref[...]) + b1_ref[...]
    he = he_ref[...].reshape(_BN * _K, _H)
    x = g1_ref[...] + _dot(he, w1b_ref[...])
    x = x.reshape(_BN, _K, _H) + a1[:, None, :]
    x = x.reshape(_BN * _K, _H)
    x = _dot(_gelu(x), w2_ref[...]) + b2_ref[...]
    x = _dot(_gelu(x), w3_ref[...]) + b3_ref[...]
    dh = jnp.sum(x.reshape(_BN, _K, _H), axis=1) * (1.0 / _SCALE)
    v = _ln(hv + dh, s1_ref[...], o1_ref[...])
    f = _dot(_gelu(_dot(v, wi_ref[...]) + bi_ref[...]), wo_ref[...]) + bo_ref[...]
    v2 = _ln(v + f, s2_ref[...], o2_ref[...])
    hv2_ref[...] = v2
    c2_ref[...] = _dot(v2, w11c_ref[...])
    a2_ref[...] = _dot(v2, w11a_ref[...]) + b11_ref[...]


def _passA(h_E, g1, h_V, W1a, b1, W1b, W2, b2, W3, b3, s1, o1, Wi, bi, Wo, bo,
           s2, o2, W11a, b11, W11c):
    nb = _N // _BN
    row = lambda i: (0, 0)
    sq = lambda i: (0, 0)
    out_sds = jax.ShapeDtypeStruct((_N, _H), jnp.float32)
    return pl.pallas_call(
        _passA_kernel,
        grid=(nb,),
        in_specs=[
            pl.BlockSpec((_BN, _K, _H), lambda i: (i, 0, 0)),
            pl.BlockSpec((_BN * _K, _H), lambda i: (i, 0)),
            pl.BlockSpec((_BN, _H), lambda i: (i, 0)),
            pl.BlockSpec((_H, _H), sq),
            pl.BlockSpec((1, _H), row),
            pl.BlockSpec((_H, _H), sq),
            pl.BlockSpec((_H, _H), sq),
            pl.BlockSpec((1, _H), row),
            pl.BlockSpec((_H, _H), sq),
            pl.BlockSpec((1, _H), row),
            pl.BlockSpec((1, _H), row),
            pl.BlockSpec((1, _H), row),
            pl.BlockSpec((_H, 4 * _H), sq),
            pl.BlockSpec((1, 4 * _H), row),
            pl.BlockSpec((4 * _H, _H), sq),
            pl.BlockSpec((1, _H), row),
            pl.BlockSpec((1, _H), row),
            pl.BlockSpec((1, _H), row),
            pl.BlockSpec((_H, _H), sq),
            pl.BlockSpec((1, _H), row),
            pl.BlockSpec((_H, _H), sq),
        ],
        out_specs=[
            pl.BlockSpec((_BN, _H), lambda i: (i, 0)),
            pl.BlockSpec((_BN, _H), lambda i: (i, 0)),
            pl.BlockSpec((_BN, _H), lambda i: (i, 0)),
        ],
        out_shape=[out_sds, out_sds, out_sds],
    )(h_E, g1, h_V, W1a, b1, W1b, W2, b2, W3, b3, s1, o1, Wi, bi, Wo, bo,
      s2, o2, W11a, b11, W11c)


def _passB_kernel(he_ref, g2_ref, a2_ref, w11b_ref, w12_ref, b12_ref, w13_ref,
                  b13_ref, s3_ref, o3_ref, out_ref):
    he = he_ref[...].reshape(_BN * _K, _H)
    x = g2_ref[...] + _dot(he, w11b_ref[...])
    x = x.reshape(_BN, _K, _H) + a2_ref[...][:, None, :]
    x = x.reshape(_BN * _K, _H)
    x = _dot(_gelu(x), w12_ref[...]) + b12_ref[...]
    x = _dot(_gelu(x), w13_ref[...]) + b13_ref[...]
    y = _ln(he + x, s3_ref[...], o3_ref[...])
    out_ref[...] = y.reshape(_BN, _K, _H)


def _passB(h_E, g2, a2, W11b, W12, b12, W13, b13, s3, o3):
    nb = _N // _BN
    row = lambda i: (0, 0)
    sq = lambda i: (0, 0)
    return pl.pallas_call(
        _passB_kernel,
        grid=(nb,),
        in_specs=[
            pl.BlockSpec((_BN, _K, _H), lambda i: (i, 0, 0)),
            pl.BlockSpec((_BN * _K, _H), lambda i: (i, 0)),
            pl.BlockSpec((_BN, _H), lambda i: (i, 0)),
            pl.BlockSpec((_H, _H), sq),
            pl.BlockSpec((_H, _H), sq),
            pl.BlockSpec((1, _H), row),
            pl.BlockSpec((_H, _H), sq),
            pl.BlockSpec((1, _H), row),
            pl.BlockSpec((1, _H), row),
            pl.BlockSpec((1, _H), row),
        ],
        out_specs=pl.BlockSpec((_BN, _K, _H), lambda i: (i, 0, 0)),
        out_shape=jax.ShapeDtypeStruct((_N, _K, _H), jnp.float32),
    )(h_E, g2, a2, W11b, W12, b12, W13, b13, s3, o3)


def kernel(h_V, h_E, E_idx, W1, b1, W2, b2, W3, b3, W11, b11, W12, b12, W13,
           b13, Wi, bi, Wo, bo, s1, o1, s2, o2, s3, o3):
    r = lambda v: v.reshape(1, -1)
    W1a, W1b, W1c = W1[:_H], W1[_H:2 * _H], W1[2 * _H:]
    W11a, W11b, W11c = W11[:_H], W11[_H:2 * _H], W11[2 * _H:]
    e_flat = E_idx.reshape(-1)

    c1 = _pre(h_V, W1c)
    g1 = _sc_gather(c1, e_flat)
    h_V2, c2, a2 = _passA(h_E, g1, h_V, W1a, r(b1), W1b, W2, r(b2), W3, r(b3),
                          r(s1), r(o1), Wi, r(bi), Wo, r(bo), r(s2), r(o2),
                          W11a, r(b11), W11c)
    g2 = _sc_gather(c2, e_flat)
    h_E_out = _passB(h_E, g2, a2, W11b, W12, r(b12), W13, r(b13), r(s3), r(o3))
    return (h_V2, h_E_out)
